# baseline (speedup 1.0000x reference)
import functools

import jax
import jax.numpy as jnp
from jax import lax
from jax.experimental import pallas as pl
from jax.experimental.pallas import tpu as pltpu
from jax.experimental.pallas import tpu_sc as plsc

N_NODES = 10000
N_EDGES = 320000
D_FEAT = 128
HIDDEN = 128
N_CLASSES = 64

NC, NS, L = 2, 16, 16
NW = NC * NS
N_PAD = 10240
CHUNK = 128
N_CHUNKS = 80
E_PAD = NW * N_CHUNKS * CHUNK
DUMP = N_NODES
HALF = N_CHUNKS // 2
RPS = N_PAD // NS


def _deg_body(dst_hbm, degp_out, idx_v, deg_v):
    c = lax.axis_index("c")
    s = lax.axis_index("s")
    wid = s * NC + c

    pltpu.sync_copy(dst_hbm.at[wid], idx_v)

    zero16 = jnp.zeros((L,), jnp.float32)

    def zero_step(i, carry):
        deg_v[pl.ds(i * L, L)] = zero16
        return carry

    lax.fori_loop(0, N_PAD // L, zero_step, 0)

    ones16 = jnp.ones((L,), jnp.float32)

    def acc_step(j, carry):
        for k in range(CHUNK // L):
            idx = idx_v[j, pl.ds(k * L, L)]
            plsc.addupdate_scatter(deg_v, [idx], ones16)
        return carry

    lax.fori_loop(0, N_CHUNKS, acc_step, 0)

    pltpu.sync_copy(deg_v, degp_out.at[wid])


def _make_deg_kernel():
    mesh = plsc.VectorSubcoreMesh(core_axis_name="c", subcore_axis_name="s")
    return pl.kernel(
        _deg_body,
        out_type=jax.ShapeDtypeStruct((NW, N_PAD), jnp.float32),
        mesh=mesh,
        compiler_params=pltpu.CompilerParams(needs_layout_passes=False),
        scratch_types=[
            pltpu.VMEM((N_CHUNKS, CHUNK), jnp.int32),
            pltpu.VMEM((N_PAD,), jnp.float32),
        ],
    )


def _make_scatter_kernel(width):
    def body(g_hbm, src_hbm, dst_hbm, out_hbm, sidx_v, didx_v, rows0_v, rows1_v,
             acc_sh, sem0, sem1, ssem0, ssem1):
        c = lax.axis_index("c")
        s = lax.axis_index("s")
        wid = s * NC + c

        zero16 = jnp.zeros((L,), jnp.float32)

        def zero_step(i, carry):
            for k in range(width // L):
                rows0_v[i, pl.ds(k * L, L)] = zero16
            return carry

        lax.fori_loop(0, CHUNK, zero_step, 0)
        for t in range(RPS // CHUNK):
            pltpu.sync_copy(
                rows0_v, acc_sh.at[pl.ds(s * RPS + t * CHUNK, CHUNK)]
            )
        plsc.subcore_barrier()

        n_pairs = HALF // 2
        for h in range(N_CHUNKS // HALF):
            pltpu.sync_copy(src_hbm.at[wid, pl.ds(h * HALF, HALF)], sidx_v)
            pltpu.sync_copy(dst_hbm.at[wid, pl.ds(h * HALF, HALF)], didx_v)
            pltpu.async_copy(g_hbm.at[sidx_v.at[0]], rows0_v, sem0)

            pltpu.async_copy(g_hbm.at[sidx_v.at[1]], rows1_v, sem1)

            def pair_step(t, carry):
                j0 = 2 * t
                pltpu.make_async_copy(
                    g_hbm.at[sidx_v.at[j0]], rows0_v, sem0
                ).wait()
                pltpu.async_copy(
                    rows0_v, acc_sh.at[didx_v.at[j0]], ssem0, add=True
                )
                pltpu.make_async_copy(
                    g_hbm.at[sidx_v.at[j0 + 1]], rows1_v, sem1
                ).wait()
                pltpu.async_copy(
                    rows1_v, acc_sh.at[didx_v.at[j0 + 1]], ssem1, add=True
                )

                @pl.when(t + 1 < n_pairs)
                def _():
                    pltpu.make_async_copy(
                        rows0_v, acc_sh.at[didx_v.at[j0]], ssem0
                    ).wait()
                    pltpu.async_copy(g_hbm.at[sidx_v.at[j0 + 2]], rows0_v, sem0)
                    pltpu.make_async_copy(
                        rows1_v, acc_sh.at[didx_v.at[j0 + 1]], ssem1
                    ).wait()
                    pltpu.async_copy(g_hbm.at[sidx_v.at[j0 + 3]], rows1_v, sem1)

                return carry

            lax.fori_loop(0, n_pairs, pair_step, 0)
            pltpu.make_async_copy(
                rows0_v, acc_sh.at[didx_v.at[HALF - 2]], ssem0
            ).wait()
            pltpu.make_async_copy(
                rows1_v, acc_sh.at[didx_v.at[HALF - 1]], ssem1
            ).wait()
        plsc.subcore_barrier()

        for t in range(RPS // CHUNK):
            start = s * RPS + t * CHUNK
            pltpu.sync_copy(acc_sh.at[pl.ds(start, CHUNK)], rows0_v)
            pltpu.sync_copy(rows0_v, out_hbm.at[c, pl.ds(start, CHUNK)])

    mesh = plsc.VectorSubcoreMesh(core_axis_name="c", subcore_axis_name="s")
    return pl.kernel(
        body,
        out_type=jax.ShapeDtypeStruct((NC, N_PAD, width), jnp.float32),
        mesh=mesh,
        compiler_params=pltpu.CompilerParams(
            needs_layout_passes=False,
            use_tc_tiling_on_sc=None if width % 128 == 0 else False,
        ),
        scratch_types=[
            pltpu.VMEM((HALF, CHUNK), jnp.int32),
            pltpu.VMEM((HALF, CHUNK), jnp.int32),
            pltpu.VMEM((CHUNK, width), jnp.float32),
            pltpu.VMEM((CHUNK, width), jnp.float32),
            pltpu.VMEM_SHARED((N_PAD, width), jnp.float32),
            pltpu.SemaphoreType.DMA,
            pltpu.SemaphoreType.DMA,
            pltpu.SemaphoreType.DMA,
            pltpu.SemaphoreType.DMA,
        ],
    )


def _dis_tc_body(degp_ref, dis_ref):
    deg = jnp.sum(degp_ref[...], axis=0, keepdims=True) + 1.0
    dis_ref[...] = lax.rsqrt(deg)


def _matmul_tc_body(x_ref, w_ref, dis_ref, hcat_ref, g1_ref):
    h = jnp.dot(x_ref[...], w_ref[...], preferred_element_type=jnp.float32)
    hcat_ref[...] = h
    g1_ref[...] = h[:, :D_FEAT] * dis_ref[...]


def _mid_tc_body(p_ref, hcat_ref, dis_ref, w2b_ref, b1_ref, g2_ref, h2_ref):
    dis = dis_ref[...]
    p = p_ref[0] + p_ref[1]
    h1 = hcat_ref[:, :D_FEAT]
    out1 = dis * p + (dis * dis) * h1 + b1_ref[...]
    x1 = jnp.maximum(out1, 0.0)
    h2 = hcat_ref[:, D_FEAT:D_FEAT + N_CLASSES] + jnp.dot(
        x1, w2b_ref[...], preferred_element_type=jnp.float32
    )
    h2_ref[...] = h2
    g2_ref[...] = dis * h2


def _final_tc_body(q_ref, h2_ref, skip_ref, dis_ref, b2_ref, out_ref):
    dis = dis_ref[...]
    q = q_ref[0] + q_ref[1]
    out_ref[...] = (
        dis * q + (dis * dis) * h2_ref[...] + b2_ref[...] + skip_ref[...]
    )


_ROW_BLK = 512
_N_BLKS = N_PAD // _ROW_BLK


def kernel(x, edge_index, W1, b1, W2, b2, W_skip):
    f32 = jnp.float32
    src = edge_index[0].astype(jnp.int32)
    dst = edge_index[1].astype(jnp.int32)
    pad = E_PAD - N_EDGES
    pad_ids = lax.iota(jnp.int32, pad)
    src_t = jnp.concatenate([src, pad_ids % N_NODES]).reshape(
        NW, N_CHUNKS, CHUNK
    )
    dst_t = jnp.concatenate([dst, DUMP + pad_ids % (N_PAD - N_NODES)]).reshape(
        NW, N_CHUNKS, CHUNK
    )
    x_pad = jnp.pad(x, ((0, N_PAD - N_NODES), (0, 0)))
    w_cat = jnp.concatenate([W1, W2[:D_FEAT], W_skip], axis=1)
    w2b = W2[D_FEAT:]
    b1r = b1.reshape(1, HIDDEN)
    b2r = b2.reshape(1, N_CLASSES)

    degp = _make_deg_kernel()(dst_t)
    dis_row = pl.pallas_call(
        _dis_tc_body,
        out_shape=jax.ShapeDtypeStruct((1, N_PAD), f32),
    )(degp)
    dis_col = dis_row.reshape(N_PAD, 1)

    wtot = HIDDEN + 2 * N_CLASSES
    hcat, g1 = pl.pallas_call(
        _matmul_tc_body,
        grid=(_N_BLKS,),
        in_specs=[
            pl.BlockSpec((_ROW_BLK, D_FEAT), lambda i: (i, 0)),
            pl.BlockSpec((D_FEAT, wtot), lambda i: (0, 0)),
            pl.BlockSpec((_ROW_BLK, 1), lambda i: (i, 0)),
        ],
        out_specs=[
            pl.BlockSpec((_ROW_BLK, wtot), lambda i: (i, 0)),
            pl.BlockSpec((_ROW_BLK, D_FEAT), lambda i: (i, 0)),
        ],
        out_shape=[
            jax.ShapeDtypeStruct((N_PAD, wtot), f32),
            jax.ShapeDtypeStruct((N_PAD, D_FEAT), f32),
        ],
    )(x_pad, w_cat, dis_col)

    p1 = _make_scatter_kernel(HIDDEN)(g1, src_t, dst_t)

    g2, h2 = pl.pallas_call(
        _mid_tc_body,
        grid=(_N_BLKS,),
        in_specs=[
            pl.BlockSpec((NC, _ROW_BLK, HIDDEN), lambda i: (0, i, 0)),
            pl.BlockSpec((_ROW_BLK, wtot), lambda i: (i, 0)),
            pl.BlockSpec((_ROW_BLK, 1), lambda i: (i, 0)),
            pl.BlockSpec((D_FEAT, N_CLASSES), lambda i: (0, 0)),
            pl.BlockSpec((1, HIDDEN), lambda i: (0, 0)),
        ],
        out_specs=[
            pl.BlockSpec((_ROW_BLK, N_CLASSES), lambda i: (i, 0)),
            pl.BlockSpec((_ROW_BLK, N_CLASSES), lambda i: (i, 0)),
        ],
        out_shape=[
            jax.ShapeDtypeStruct((N_PAD, N_CLASSES), f32),
            jax.ShapeDtypeStruct((N_PAD, N_CLASSES), f32),
        ],
    )(p1, hcat, dis_col, w2b, b1r)

    q2 = _make_scatter_kernel(N_CLASSES)(g2, src_t, dst_t)

    out = pl.pallas_call(
        _final_tc_body,
        grid=(_N_BLKS,),
        in_specs=[
            pl.BlockSpec((NC, _ROW_BLK, N_CLASSES), lambda i: (0, i, 0)),
            pl.BlockSpec((_ROW_BLK, N_CLASSES), lambda i: (i, 0)),
            pl.BlockSpec((_ROW_BLK, N_CLASSES), lambda i: (i, 0)),
            pl.BlockSpec((_ROW_BLK, 1), lambda i: (i, 0)),
            pl.BlockSpec((1, N_CLASSES), lambda i: (0, 0)),
        ],
        out_specs=pl.BlockSpec((_ROW_BLK, N_CLASSES), lambda i: (i, 0)),
        out_shape=jax.ShapeDtypeStruct((N_PAD, N_CLASSES), f32),
    )(q2, h2, hcat[:, HIDDEN + N_CLASSES:], dis_col, b2r)

    return out[:N_NODES]


# speedup vs baseline: 1.1567x; 1.1567x over previous
import functools

import jax
import jax.numpy as jnp
from jax import lax
from jax.experimental import pallas as pl
from jax.experimental.pallas import tpu as pltpu
from jax.experimental.pallas import tpu_sc as plsc

N_NODES = 10000
N_EDGES = 320000
D_FEAT = 128
HIDDEN = 128
N_CLASSES = 64

NC, NS, L = 2, 16, 16
NW = NC * NS
N_PAD = 10240
CHUNK = 128
N_CHUNKS = 80
E_PAD = NW * N_CHUNKS * CHUNK
DUMP = N_NODES
HALF = N_CHUNKS // 2
RPS = N_PAD // NS


def _deg_body(dst_hbm, degp_out, idx_v, deg_v):
    c = lax.axis_index("c")
    s = lax.axis_index("s")
    wid = s * NC + c

    pltpu.sync_copy(dst_hbm.at[wid], idx_v)

    zero16 = jnp.zeros((L,), jnp.float32)

    def zero_step(i, carry):
        deg_v[pl.ds(i * L, L)] = zero16
        return carry

    lax.fori_loop(0, N_PAD // L, zero_step, 0)

    ones16 = jnp.ones((L,), jnp.float32)

    def acc_step(j, carry):
        for k in range(CHUNK // L):
            idx = idx_v[j, pl.ds(k * L, L)]
            plsc.addupdate_scatter(deg_v, [idx], ones16)
        return carry

    lax.fori_loop(0, N_CHUNKS, acc_step, 0)

    pltpu.sync_copy(deg_v, degp_out.at[wid])


def _make_deg_kernel():
    mesh = plsc.VectorSubcoreMesh(core_axis_name="c", subcore_axis_name="s")
    return pl.kernel(
        _deg_body,
        out_type=jax.ShapeDtypeStruct((NW, N_PAD), jnp.float32),
        mesh=mesh,
        compiler_params=pltpu.CompilerParams(needs_layout_passes=False),
        scratch_types=[
            pltpu.VMEM((N_CHUNKS, CHUNK), jnp.int32),
            pltpu.VMEM((N_PAD,), jnp.float32),
        ],
    )


def _make_scatter_kernel(width):
    def body(g_hbm, src_hbm, dst_hbm, out_hbm, sidx_v, didx_v, rows0_v, rows1_v,
             acc_sh, sem0, sem1):
        c = lax.axis_index("c")
        s = lax.axis_index("s")
        wid = s * NC + c

        zero16 = jnp.zeros((L,), jnp.float32)

        def zero_step(i, carry):
            for k in range(width // L):
                rows0_v[i, pl.ds(k * L, L)] = zero16
            return carry

        lax.fori_loop(0, CHUNK, zero_step, 0)
        for t in range(RPS // CHUNK):
            pltpu.sync_copy(
                rows0_v, acc_sh.at[pl.ds(s * RPS + t * CHUNK, CHUNK)]
            )
        plsc.subcore_barrier()

        n_pairs = HALF // 2
        for h in range(N_CHUNKS // HALF):
            pltpu.sync_copy(src_hbm.at[wid, pl.ds(h * HALF, HALF)], sidx_v)
            pltpu.sync_copy(dst_hbm.at[wid, pl.ds(h * HALF, HALF)], didx_v)
            pltpu.async_copy(g_hbm.at[sidx_v.at[0]], rows0_v, sem0)

            def pair_step(t, carry):
                j0 = 2 * t
                pltpu.async_copy(g_hbm.at[sidx_v.at[j0 + 1]], rows1_v, sem1)
                pltpu.make_async_copy(
                    g_hbm.at[sidx_v.at[j0]], rows0_v, sem0
                ).wait()
                pltpu.sync_copy(rows0_v, acc_sh.at[didx_v.at[j0]], add=True)

                @pl.when(t + 1 < n_pairs)
                def _():
                    pltpu.async_copy(g_hbm.at[sidx_v.at[j0 + 2]], rows0_v, sem0)

                pltpu.make_async_copy(
                    g_hbm.at[sidx_v.at[j0 + 1]], rows1_v, sem1
                ).wait()
                pltpu.sync_copy(rows1_v, acc_sh.at[didx_v.at[j0 + 1]], add=True)
                return carry

            lax.fori_loop(0, n_pairs, pair_step, 0)
        plsc.subcore_barrier()

        for t in range(RPS // CHUNK):
            start = s * RPS + t * CHUNK
            pltpu.sync_copy(acc_sh.at[pl.ds(start, CHUNK)], rows0_v)
            pltpu.sync_copy(rows0_v, out_hbm.at[c, pl.ds(start, CHUNK)])

    mesh = plsc.VectorSubcoreMesh(core_axis_name="c", subcore_axis_name="s")
    return pl.kernel(
        body,
        out_type=jax.ShapeDtypeStruct((NC, N_PAD, width), jnp.float32),
        mesh=mesh,
        compiler_params=pltpu.CompilerParams(
            needs_layout_passes=False,
            use_tc_tiling_on_sc=None if width % 128 == 0 else False,
        ),
        scratch_types=[
            pltpu.VMEM((HALF, CHUNK), jnp.int32),
            pltpu.VMEM((HALF, CHUNK), jnp.int32),
            pltpu.VMEM((CHUNK, width), jnp.float32),
            pltpu.VMEM((CHUNK, width), jnp.float32),
            pltpu.VMEM_SHARED((N_PAD, width), jnp.float32),
            pltpu.SemaphoreType.DMA,
            pltpu.SemaphoreType.DMA,
        ],
    )


def _dis_tc_body(degp_ref, dis_ref):
    deg = jnp.sum(degp_ref[...], axis=0, keepdims=True) + 1.0
    dis_ref[...] = lax.rsqrt(deg)


def _matmul_tc_body(x_ref, w_ref, dis_ref, hcat_ref, g1_ref):
    h = jnp.dot(x_ref[...], w_ref[...], preferred_element_type=jnp.float32)
    hcat_ref[...] = h
    g1_ref[...] = h[:, :D_FEAT] * dis_ref[...]


def _mid_tc_body(p_ref, hcat_ref, dis_ref, w2b_ref, b1_ref, g2_ref, h2_ref):
    dis = dis_ref[...]
    p = p_ref[0] + p_ref[1]
    h1 = hcat_ref[:, :D_FEAT]
    out1 = dis * p + (dis * dis) * h1 + b1_ref[...]
    x1 = jnp.maximum(out1, 0.0)
    h2 = hcat_ref[:, D_FEAT:D_FEAT + N_CLASSES] + jnp.dot(
        x1, w2b_ref[...], preferred_element_type=jnp.float32
    )
    h2_ref[...] = h2
    g2_ref[...] = dis * h2


def _final_tc_body(q_ref, h2_ref, hcat_ref, dis_ref, b2_ref, out_ref):
    dis = dis_ref[...]
    q = q_ref[0] + q_ref[1]
    skip = hcat_ref[:, HIDDEN + N_CLASSES:]
    out_ref[...] = dis * q + (dis * dis) * h2_ref[...] + b2_ref[...] + skip


_ROW_BLK = 400
_N_BLKS = N_NODES // _ROW_BLK


def kernel(x, edge_index, W1, b1, W2, b2, W_skip):
    f32 = jnp.float32
    src = edge_index[0].astype(jnp.int32)
    dst = edge_index[1].astype(jnp.int32)
    pad = E_PAD - N_EDGES
    pad_ids = lax.iota(jnp.int32, pad)
    src_t = jnp.concatenate([src, pad_ids % N_NODES]).reshape(
        NW, N_CHUNKS, CHUNK
    )
    dst_t = jnp.concatenate([dst, DUMP + pad_ids % (N_PAD - N_NODES)]).reshape(
        NW, N_CHUNKS, CHUNK
    )
    w_cat = jnp.concatenate([W1, W2[:D_FEAT], W_skip], axis=1)
    w2b = W2[D_FEAT:]
    b1r = b1.reshape(1, HIDDEN)
    b2r = b2.reshape(1, N_CLASSES)

    degp = _make_deg_kernel()(dst_t)
    dis_row = pl.pallas_call(
        _dis_tc_body,
        out_shape=jax.ShapeDtypeStruct((1, N_PAD), f32),
    )(degp)
    dis_col = dis_row.reshape(N_PAD, 1)[:N_NODES]

    wtot = HIDDEN + 2 * N_CLASSES
    hcat, g1 = pl.pallas_call(
        _matmul_tc_body,
        grid=(_N_BLKS,),
        in_specs=[
            pl.BlockSpec((_ROW_BLK, D_FEAT), lambda i: (i, 0)),
            pl.BlockSpec((D_FEAT, wtot), lambda i: (0, 0)),
            pl.BlockSpec((_ROW_BLK, 1), lambda i: (i, 0)),
        ],
        out_specs=[
            pl.BlockSpec((_ROW_BLK, wtot), lambda i: (i, 0)),
            pl.BlockSpec((_ROW_BLK, D_FEAT), lambda i: (i, 0)),
        ],
        out_shape=[
            jax.ShapeDtypeStruct((N_NODES, wtot), f32),
            jax.ShapeDtypeStruct((N_NODES, D_FEAT), f32),
        ],
    )(x, w_cat, dis_col)

    p1 = _make_scatter_kernel(HIDDEN)(g1, src_t, dst_t)

    g2, h2 = pl.pallas_call(
        _mid_tc_body,
        grid=(_N_BLKS,),
        in_specs=[
            pl.BlockSpec((NC, _ROW_BLK, HIDDEN), lambda i: (0, i, 0)),
            pl.BlockSpec((_ROW_BLK, wtot), lambda i: (i, 0)),
            pl.BlockSpec((_ROW_BLK, 1), lambda i: (i, 0)),
            pl.BlockSpec((D_FEAT, N_CLASSES), lambda i: (0, 0)),
            pl.BlockSpec((1, HIDDEN), lambda i: (0, 0)),
        ],
        out_specs=[
            pl.BlockSpec((_ROW_BLK, N_CLASSES), lambda i: (i, 0)),
            pl.BlockSpec((_ROW_BLK, N_CLASSES), lambda i: (i, 0)),
        ],
        out_shape=[
            jax.ShapeDtypeStruct((N_NODES, N_CLASSES), f32),
            jax.ShapeDtypeStruct((N_NODES, N_CLASSES), f32),
        ],
    )(p1, hcat, dis_col, w2b, b1r)

    q2 = _make_scatter_kernel(N_CLASSES)(g2, src_t, dst_t)

    out = pl.pallas_call(
        _final_tc_body,
        grid=(_N_BLKS,),
        in_specs=[
            pl.BlockSpec((NC, _ROW_BLK, N_CLASSES), lambda i: (0, i, 0)),
            pl.BlockSpec((_ROW_BLK, N_CLASSES), lambda i: (i, 0)),
            pl.BlockSpec((_ROW_BLK, wtot), lambda i: (i, 0)),
            pl.BlockSpec((_ROW_BLK, 1), lambda i: (i, 0)),
            pl.BlockSpec((1, N_CLASSES), lambda i: (0, 0)),
        ],
        out_specs=pl.BlockSpec((_ROW_BLK, N_CLASSES), lambda i: (i, 0)),
        out_shape=jax.ShapeDtypeStruct((N_NODES, N_CLASSES), f32),
    )(q2, h2, hcat, dis_col, b2r)

    return out


# speedup vs baseline: 1.2727x; 1.1003x over previous
import functools

import jax
import jax.numpy as jnp
from jax import lax
from jax.experimental import pallas as pl
from jax.experimental.pallas import tpu as pltpu
from jax.experimental.pallas import tpu_sc as plsc

N_NODES = 10000
N_EDGES = 320000
D_FEAT = 128
HIDDEN = 128
N_CLASSES = 64

NC, NS, L = 2, 16, 16
NW = NC * NS
N_PAD = 10240
CHUNK = 128
N_CHUNKS = 80
E_PAD = NW * N_CHUNKS * CHUNK
DUMP = N_NODES
HALF = N_CHUNKS // 2
RPS = N_PAD // NS


def _deg_body(dst_hbm, degp_out, idx_v, deg_v):
    c = lax.axis_index("c")
    s = lax.axis_index("s")
    wid = s * NC + c

    pltpu.sync_copy(dst_hbm.at[wid], idx_v)

    zero16 = jnp.zeros((L,), jnp.float32)

    def zero_step(i, carry):
        deg_v[pl.ds(i * L, L)] = zero16
        return carry

    lax.fori_loop(0, N_PAD // L, zero_step, 0)

    ones16 = jnp.ones((L,), jnp.float32)

    def acc_step(j, carry):
        for k in range(CHUNK // L):
            idx = idx_v[j, pl.ds(k * L, L)]
            plsc.addupdate_scatter(deg_v, [idx], ones16)
        return carry

    lax.fori_loop(0, N_CHUNKS, acc_step, 0)

    pltpu.sync_copy(deg_v, degp_out.at[wid])


def _make_deg_kernel():
    mesh = plsc.VectorSubcoreMesh(core_axis_name="c", subcore_axis_name="s")
    return pl.kernel(
        _deg_body,
        out_type=jax.ShapeDtypeStruct((NW, N_PAD), jnp.float32),
        mesh=mesh,
        compiler_params=pltpu.CompilerParams(needs_layout_passes=False),
        scratch_types=[
            pltpu.VMEM((N_CHUNKS, CHUNK), jnp.int32),
            pltpu.VMEM((N_PAD,), jnp.float32),
        ],
    )


def _make_scatter_kernel(width):
    def body(g_hbm, src_hbm, dst_hbm, out_hbm, sidx_v, didx_v, rows0_v, rows1_v,
             acc_sh, sem0, sem1):
        c = lax.axis_index("c")
        s = lax.axis_index("s")
        wid = s * NC + c

        zero16 = jnp.zeros((L,), jnp.float32)

        def zero_step(i, carry):
            for k in range(width // L):
                rows0_v[i, pl.ds(k * L, L)] = zero16
            return carry

        lax.fori_loop(0, CHUNK, zero_step, 0)
        for t in range(RPS // CHUNK):
            pltpu.sync_copy(
                rows0_v, acc_sh.at[pl.ds(s * RPS + t * CHUNK, CHUNK)]
            )
        plsc.subcore_barrier()

        n_pairs = HALF // 2
        for h in range(N_CHUNKS // HALF):
            pltpu.sync_copy(src_hbm.at[wid, pl.ds(h * HALF, HALF)], sidx_v)
            pltpu.sync_copy(dst_hbm.at[wid, pl.ds(h * HALF, HALF)], didx_v)
            pltpu.async_copy(g_hbm.at[sidx_v.at[0]], rows0_v, sem0)

            def pair_step(t, carry):
                j0 = 2 * t
                pltpu.async_copy(g_hbm.at[sidx_v.at[j0 + 1]], rows1_v, sem1)
                pltpu.make_async_copy(
                    g_hbm.at[sidx_v.at[j0]], rows0_v, sem0
                ).wait()
                pltpu.sync_copy(rows0_v, acc_sh.at[didx_v.at[j0]], add=True)

                @pl.when(t + 1 < n_pairs)
                def _():
                    pltpu.async_copy(g_hbm.at[sidx_v.at[j0 + 2]], rows0_v, sem0)

                pltpu.make_async_copy(
                    g_hbm.at[sidx_v.at[j0 + 1]], rows1_v, sem1
                ).wait()
                pltpu.sync_copy(rows1_v, acc_sh.at[didx_v.at[j0 + 1]], add=True)
                return carry

            lax.fori_loop(0, n_pairs, pair_step, 0)
        plsc.subcore_barrier()

        for t in range(RPS // CHUNK):
            start = s * RPS + t * CHUNK
            pltpu.sync_copy(acc_sh.at[pl.ds(start, CHUNK)], rows0_v)
            pltpu.sync_copy(rows0_v, out_hbm.at[c, pl.ds(start, CHUNK)])

    mesh = plsc.VectorSubcoreMesh(core_axis_name="c", subcore_axis_name="s")
    return pl.kernel(
        body,
        out_type=jax.ShapeDtypeStruct((NC, N_PAD, width), jnp.float32),
        mesh=mesh,
        compiler_params=pltpu.CompilerParams(
            needs_layout_passes=False,
            use_tc_tiling_on_sc=None if width % 128 == 0 else False,
        ),
        scratch_types=[
            pltpu.VMEM((HALF, CHUNK), jnp.int32),
            pltpu.VMEM((HALF, CHUNK), jnp.int32),
            pltpu.VMEM((CHUNK, width), jnp.float32),
            pltpu.VMEM((CHUNK, width), jnp.float32),
            pltpu.VMEM_SHARED((N_PAD, width), jnp.float32),
            pltpu.SemaphoreType.DMA,
            pltpu.SemaphoreType.DMA,
        ],
    )


def _dis_tc_body(degp_ref, dis_ref):
    deg = jnp.sum(degp_ref[...], axis=0, keepdims=True) + 1.0
    dis_ref[...] = lax.rsqrt(deg)


def _matmul_tc_body(x_ref, w_ref, dis_ref, hcat_ref, g1_ref):
    h = jnp.dot(x_ref[...], w_ref[...], preferred_element_type=jnp.float32)
    hcat_ref[...] = h
    g1_ref[...] = h[:, :D_FEAT] * dis_ref[...]


def _mid_tc_body(p_ref, hcat_ref, dis_ref, w2b_ref, b1_ref, g2_ref, h2_ref):
    dis = dis_ref[...]
    p = p_ref[0] + p_ref[1]
    h1 = hcat_ref[:, :D_FEAT]
    out1 = dis * p + (dis * dis) * h1 + b1_ref[...]
    x1 = jnp.maximum(out1, 0.0)
    h2 = hcat_ref[:, D_FEAT:D_FEAT + N_CLASSES] + jnp.dot(
        x1, w2b_ref[...], preferred_element_type=jnp.float32
    )
    h2_ref[...] = h2
    g2_ref[...] = dis * h2


def _final_tc_body(q_ref, h2_ref, hcat_ref, dis_ref, b2_ref, out_ref):
    dis = dis_ref[...]
    q = q_ref[0] + q_ref[1]
    skip = hcat_ref[:, HIDDEN + N_CLASSES:]
    out_ref[...] = dis * q + (dis * dis) * h2_ref[...] + b2_ref[...] + skip


_ROW_BLK = 2000
_N_BLKS = N_NODES // _ROW_BLK


def kernel(x, edge_index, W1, b1, W2, b2, W_skip):
    f32 = jnp.float32
    src = edge_index[0].astype(jnp.int32)
    dst = edge_index[1].astype(jnp.int32)
    pad = E_PAD - N_EDGES
    pad_ids = lax.iota(jnp.int32, pad)
    src_t = jnp.concatenate([src, pad_ids % N_NODES]).reshape(
        NW, N_CHUNKS, CHUNK
    )
    dst_t = jnp.concatenate([dst, DUMP + pad_ids % (N_PAD - N_NODES)]).reshape(
        NW, N_CHUNKS, CHUNK
    )
    w_cat = jnp.concatenate([W1, W2[:D_FEAT], W_skip], axis=1)
    w2b = W2[D_FEAT:]
    b1r = b1.reshape(1, HIDDEN)
    b2r = b2.reshape(1, N_CLASSES)

    degp = _make_deg_kernel()(dst_t)
    dis_row = pl.pallas_call(
        _dis_tc_body,
        out_shape=jax.ShapeDtypeStruct((1, N_PAD), f32),
    )(degp)
    dis_col = dis_row.reshape(N_PAD, 1)[:N_NODES]

    wtot = HIDDEN + 2 * N_CLASSES
    hcat, g1 = pl.pallas_call(
        _matmul_tc_body,
        grid=(_N_BLKS,),
        in_specs=[
            pl.BlockSpec((_ROW_BLK, D_FEAT), lambda i: (i, 0)),
            pl.BlockSpec((D_FEAT, wtot), lambda i: (0, 0)),
            pl.BlockSpec((_ROW_BLK, 1), lambda i: (i, 0)),
        ],
        out_specs=[
            pl.BlockSpec((_ROW_BLK, wtot), lambda i: (i, 0)),
            pl.BlockSpec((_ROW_BLK, D_FEAT), lambda i: (i, 0)),
        ],
        out_shape=[
            jax.ShapeDtypeStruct((N_NODES, wtot), f32),
            jax.ShapeDtypeStruct((N_NODES, D_FEAT), f32),
        ],
    )(x, w_cat, dis_col)

    p1 = _make_scatter_kernel(HIDDEN)(g1, src_t, dst_t)

    g2, h2 = pl.pallas_call(
        _mid_tc_body,
        grid=(_N_BLKS,),
        in_specs=[
            pl.BlockSpec((NC, _ROW_BLK, HIDDEN), lambda i: (0, i, 0)),
            pl.BlockSpec((_ROW_BLK, wtot), lambda i: (i, 0)),
            pl.BlockSpec((_ROW_BLK, 1), lambda i: (i, 0)),
            pl.BlockSpec((D_FEAT, N_CLASSES), lambda i: (0, 0)),
            pl.BlockSpec((1, HIDDEN), lambda i: (0, 0)),
        ],
        out_specs=[
            pl.BlockSpec((_ROW_BLK, N_CLASSES), lambda i: (i, 0)),
            pl.BlockSpec((_ROW_BLK, N_CLASSES), lambda i: (i, 0)),
        ],
        out_shape=[
            jax.ShapeDtypeStruct((N_NODES, N_CLASSES), f32),
            jax.ShapeDtypeStruct((N_NODES, N_CLASSES), f32),
        ],
    )(p1, hcat, dis_col, w2b, b1r)

    q2 = _make_scatter_kernel(N_CLASSES)(g2, src_t, dst_t)

    out = pl.pallas_call(
        _final_tc_body,
        grid=(_N_BLKS,),
        in_specs=[
            pl.BlockSpec((NC, _ROW_BLK, N_CLASSES), lambda i: (0, i, 0)),
            pl.BlockSpec((_ROW_BLK, N_CLASSES), lambda i: (i, 0)),
            pl.BlockSpec((_ROW_BLK, wtot), lambda i: (i, 0)),
            pl.BlockSpec((_ROW_BLK, 1), lambda i: (i, 0)),
            pl.BlockSpec((1, N_CLASSES), lambda i: (0, 0)),
        ],
        out_specs=pl.BlockSpec((_ROW_BLK, N_CLASSES), lambda i: (i, 0)),
        out_shape=jax.ShapeDtypeStruct((N_NODES, N_CLASSES), f32),
    )(q2, h2, hcat, dis_col, b2r)

    return out


# speedup vs baseline: 1.2753x; 1.0020x over previous
import functools

import jax
import jax.numpy as jnp
from jax import lax
from jax.experimental import pallas as pl
from jax.experimental.pallas import tpu as pltpu
from jax.experimental.pallas import tpu_sc as plsc

N_NODES = 10000
N_EDGES = 320000
D_FEAT = 128
HIDDEN = 128
N_CLASSES = 64

NC, NS, L = 2, 16, 16
NW = NC * NS
N_PAD = 10240
CHUNK = 128
N_CHUNKS = 80
E_PAD = NW * N_CHUNKS * CHUNK
DUMP = N_NODES
HALF = N_CHUNKS // 2
RPS = N_PAD // NS


def _deg_body(dst_hbm, degp_out, idx_v, deg_v):
    c = lax.axis_index("c")
    s = lax.axis_index("s")
    wid = s * NC + c

    pltpu.sync_copy(dst_hbm.at[wid], idx_v)

    zero16 = jnp.zeros((L,), jnp.float32)

    def zero_step(i, carry):
        deg_v[pl.ds(i * L, L)] = zero16
        return carry

    lax.fori_loop(0, N_PAD // L, zero_step, 0)

    ones16 = jnp.ones((L,), jnp.float32)

    def acc_step(j, carry):
        for k in range(CHUNK // L):
            idx = idx_v[j, pl.ds(k * L, L)]
            plsc.addupdate_scatter(deg_v, [idx], ones16)
        return carry

    lax.fori_loop(0, N_CHUNKS, acc_step, 0)

    pltpu.sync_copy(deg_v, degp_out.at[wid])


def _make_deg_kernel():
    mesh = plsc.VectorSubcoreMesh(core_axis_name="c", subcore_axis_name="s")
    return pl.kernel(
        _deg_body,
        out_type=jax.ShapeDtypeStruct((NW, N_PAD), jnp.float32),
        mesh=mesh,
        compiler_params=pltpu.CompilerParams(needs_layout_passes=False),
        scratch_types=[
            pltpu.VMEM((N_CHUNKS, CHUNK), jnp.int32),
            pltpu.VMEM((N_PAD,), jnp.float32),
        ],
    )


def _make_scatter_kernel(width):
    def body(g_hbm, src_hbm, dst_hbm, out_hbm, sidx_v, didx_v, rows0_v, rows1_v,
             acc_sh, sem0, sem1):
        c = lax.axis_index("c")
        s = lax.axis_index("s")
        wid = s * NC + c

        zero16 = jnp.zeros((L,), jnp.float32)

        def zero_step(i, carry):
            for k in range(width // L):
                rows0_v[i, pl.ds(k * L, L)] = zero16
            return carry

        lax.fori_loop(0, CHUNK, zero_step, 0)
        for t in range(RPS // CHUNK):
            pltpu.sync_copy(
                rows0_v, acc_sh.at[pl.ds(s * RPS + t * CHUNK, CHUNK)]
            )
        plsc.subcore_barrier()

        n_pairs = HALF // 2
        for h in range(N_CHUNKS // HALF):
            pltpu.sync_copy(src_hbm.at[wid, pl.ds(h * HALF, HALF)], sidx_v)
            pltpu.sync_copy(dst_hbm.at[wid, pl.ds(h * HALF, HALF)], didx_v)
            pltpu.async_copy(g_hbm.at[sidx_v.at[0]], rows0_v, sem0)

            def pair_step(t, carry):
                j0 = 2 * t
                pltpu.async_copy(g_hbm.at[sidx_v.at[j0 + 1]], rows1_v, sem1)
                pltpu.make_async_copy(
                    g_hbm.at[sidx_v.at[j0]], rows0_v, sem0
                ).wait()
                pltpu.sync_copy(rows0_v, acc_sh.at[didx_v.at[j0]], add=True)

                @pl.when(t + 1 < n_pairs)
                def _():
                    pltpu.async_copy(g_hbm.at[sidx_v.at[j0 + 2]], rows0_v, sem0)

                pltpu.make_async_copy(
                    g_hbm.at[sidx_v.at[j0 + 1]], rows1_v, sem1
                ).wait()
                pltpu.sync_copy(rows1_v, acc_sh.at[didx_v.at[j0 + 1]], add=True)
                return carry

            lax.fori_loop(0, n_pairs, pair_step, 0)
        plsc.subcore_barrier()

        start = s * RPS
        pltpu.sync_copy(
            acc_sh.at[pl.ds(start, RPS)], out_hbm.at[c, pl.ds(start, RPS)]
        )

    mesh = plsc.VectorSubcoreMesh(core_axis_name="c", subcore_axis_name="s")
    return pl.kernel(
        body,
        out_type=jax.ShapeDtypeStruct((NC, N_PAD, width), jnp.float32),
        mesh=mesh,
        compiler_params=pltpu.CompilerParams(
            needs_layout_passes=False,
            use_tc_tiling_on_sc=None if width % 128 == 0 else False,
        ),
        scratch_types=[
            pltpu.VMEM((HALF, CHUNK), jnp.int32),
            pltpu.VMEM((HALF, CHUNK), jnp.int32),
            pltpu.VMEM((CHUNK, width), jnp.float32),
            pltpu.VMEM((CHUNK, width), jnp.float32),
            pltpu.VMEM_SHARED((N_PAD, width), jnp.float32),
            pltpu.SemaphoreType.DMA,
            pltpu.SemaphoreType.DMA,
        ],
    )


def _dis_tc_body(degp_ref, dis_ref):
    deg = jnp.sum(degp_ref[...], axis=0, keepdims=True) + 1.0
    dis_ref[...] = lax.rsqrt(deg)


def _matmul_tc_body(x_ref, w_ref, dis_ref, hcat_ref, g1_ref, xskip_ref):
    h = jnp.dot(x_ref[...], w_ref[...], preferred_element_type=jnp.float32)
    hcat_ref[...] = h[:, :D_FEAT + N_CLASSES]
    xskip_ref[...] = h[:, D_FEAT + N_CLASSES:]
    g1_ref[...] = h[:, :D_FEAT] * dis_ref[...]


def _mid_tc_body(p_ref, hcat_ref, dis_ref, w2b_ref, b1_ref, g2_ref, h2_ref):
    dis = dis_ref[...]
    p = p_ref[0] + p_ref[1]
    h1 = hcat_ref[:, :D_FEAT]
    out1 = dis * p + (dis * dis) * h1 + b1_ref[...]
    x1 = jnp.maximum(out1, 0.0)
    h2 = hcat_ref[:, D_FEAT:D_FEAT + N_CLASSES] + jnp.dot(
        x1, w2b_ref[...], preferred_element_type=jnp.float32
    )
    h2_ref[...] = h2
    g2_ref[...] = dis * h2


def _final_tc_body(q_ref, h2_ref, xskip_ref, dis_ref, b2_ref, out_ref):
    dis = dis_ref[...]
    q = q_ref[0] + q_ref[1]
    out_ref[...] = (
        dis * q + (dis * dis) * h2_ref[...] + b2_ref[...] + xskip_ref[...]
    )


_ROW_BLK = 2000
_N_BLKS = N_NODES // _ROW_BLK


def kernel(x, edge_index, W1, b1, W2, b2, W_skip):
    f32 = jnp.float32
    src = edge_index[0].astype(jnp.int32)
    dst = edge_index[1].astype(jnp.int32)
    pad = E_PAD - N_EDGES
    pad_ids = lax.iota(jnp.int32, pad)
    src_t = jnp.concatenate([src, pad_ids % N_NODES]).reshape(
        NW, N_CHUNKS, CHUNK
    )
    dst_t = jnp.concatenate([dst, DUMP + pad_ids % (N_PAD - N_NODES)]).reshape(
        NW, N_CHUNKS, CHUNK
    )
    w_cat = jnp.concatenate([W1, W2[:D_FEAT], W_skip], axis=1)
    w2b = W2[D_FEAT:]
    b1r = b1.reshape(1, HIDDEN)
    b2r = b2.reshape(1, N_CLASSES)

    degp = _make_deg_kernel()(dst_t)
    dis_row = pl.pallas_call(
        _dis_tc_body,
        out_shape=jax.ShapeDtypeStruct((1, N_PAD), f32),
    )(degp)
    dis_col = dis_row.reshape(N_PAD, 1)[:N_NODES]

    wtot = HIDDEN + 2 * N_CLASSES
    hcat, g1, xskip = pl.pallas_call(
        _matmul_tc_body,
        grid=(_N_BLKS,),
        in_specs=[
            pl.BlockSpec((_ROW_BLK, D_FEAT), lambda i: (i, 0)),
            pl.BlockSpec((D_FEAT, wtot), lambda i: (0, 0)),
            pl.BlockSpec((_ROW_BLK, 1), lambda i: (i, 0)),
        ],
        out_specs=[
            pl.BlockSpec((_ROW_BLK, HIDDEN + N_CLASSES), lambda i: (i, 0)),
            pl.BlockSpec((_ROW_BLK, D_FEAT), lambda i: (i, 0)),
            pl.BlockSpec((_ROW_BLK, N_CLASSES), lambda i: (i, 0)),
        ],
        out_shape=[
            jax.ShapeDtypeStruct((N_NODES, HIDDEN + N_CLASSES), f32),
            jax.ShapeDtypeStruct((N_NODES, D_FEAT), f32),
            jax.ShapeDtypeStruct((N_NODES, N_CLASSES), f32),
        ],
    )(x, w_cat, dis_col)

    p1 = _make_scatter_kernel(HIDDEN)(g1, src_t, dst_t)

    g2, h2 = pl.pallas_call(
        _mid_tc_body,
        grid=(_N_BLKS,),
        in_specs=[
            pl.BlockSpec((NC, _ROW_BLK, HIDDEN), lambda i: (0, i, 0)),
            pl.BlockSpec((_ROW_BLK, HIDDEN + N_CLASSES), lambda i: (i, 0)),
            pl.BlockSpec((_ROW_BLK, 1), lambda i: (i, 0)),
            pl.BlockSpec((D_FEAT, N_CLASSES), lambda i: (0, 0)),
            pl.BlockSpec((1, HIDDEN), lambda i: (0, 0)),
        ],
        out_specs=[
            pl.BlockSpec((_ROW_BLK, N_CLASSES), lambda i: (i, 0)),
            pl.BlockSpec((_ROW_BLK, N_CLASSES), lambda i: (i, 0)),
        ],
        out_shape=[
            jax.ShapeDtypeStruct((N_NODES, N_CLASSES), f32),
            jax.ShapeDtypeStruct((N_NODES, N_CLASSES), f32),
        ],
    )(p1, hcat, dis_col, w2b, b1r)

    q2 = _make_scatter_kernel(N_CLASSES)(g2, src_t, dst_t)

    out = pl.pallas_call(
        _final_tc_body,
        grid=(_N_BLKS,),
        in_specs=[
            pl.BlockSpec((NC, _ROW_BLK, N_CLASSES), lambda i: (0, i, 0)),
            pl.BlockSpec((_ROW_BLK, N_CLASSES), lambda i: (i, 0)),
            pl.BlockSpec((_ROW_BLK, N_CLASSES), lambda i: (i, 0)),
            pl.BlockSpec((_ROW_BLK, 1), lambda i: (i, 0)),
            pl.BlockSpec((1, N_CLASSES), lambda i: (0, 0)),
        ],
        out_specs=pl.BlockSpec((_ROW_BLK, N_CLASSES), lambda i: (i, 0)),
        out_shape=jax.ShapeDtypeStruct((N_NODES, N_CLASSES), f32),
    )(q2, h2, xskip, dis_col, b2r)

    return out


# speedup vs baseline: 1.3478x; 1.0569x over previous
import functools

import jax
import jax.numpy as jnp
from jax import lax
from jax.experimental import pallas as pl
from jax.experimental.pallas import tpu as pltpu
from jax.experimental.pallas import tpu_sc as plsc

N_NODES = 10000
N_EDGES = 320000
D_FEAT = 128
HIDDEN = 128
N_CLASSES = 64

NC, NS, L = 2, 16, 16
NW = NC * NS
N_PAD = 10240
CHUNK = 128
N_CHUNKS = 80
E_PAD = NW * N_CHUNKS * CHUNK
DUMP = N_NODES
HALF = N_CHUNKS // 2
RPS = N_PAD // NS


def _deg_body(dst_hbm, degp_out, idx_v, deg_v):
    c = lax.axis_index("c")
    s = lax.axis_index("s")
    wid = s * NC + c

    pltpu.sync_copy(dst_hbm.at[wid], idx_v)

    zero16 = jnp.zeros((L,), jnp.float32)

    def zero_step(i, carry):
        deg_v[pl.ds(i * L, L)] = zero16
        return carry

    lax.fori_loop(0, N_PAD // L, zero_step, 0)

    ones16 = jnp.ones((L,), jnp.float32)

    def acc_step(j, carry):
        for k in range(CHUNK // L):
            idx = idx_v[j, pl.ds(k * L, L)]
            plsc.addupdate_scatter(deg_v, [idx], ones16)
        return carry

    lax.fori_loop(0, N_CHUNKS, acc_step, 0)

    pltpu.sync_copy(deg_v, degp_out.at[wid])


def _make_deg_kernel():
    mesh = plsc.VectorSubcoreMesh(core_axis_name="c", subcore_axis_name="s")
    return pl.kernel(
        _deg_body,
        out_type=jax.ShapeDtypeStruct((NW, N_PAD), jnp.float32),
        mesh=mesh,
        compiler_params=pltpu.CompilerParams(needs_layout_passes=False),
        scratch_types=[
            pltpu.VMEM((N_CHUNKS, CHUNK), jnp.int32),
            pltpu.VMEM((N_PAD,), jnp.float32),
        ],
    )


def _make_scatter_kernel(width):
    nbuf = 2 if width % 128 == 0 else 4

    def body(g_hbm, src_hbm, dst_hbm, out_hbm, sidx_v, didx_v, *rest):
        rows = rest[:nbuf]
        acc_sh = rest[nbuf]
        sems = rest[nbuf + 1:]
        rows0_v = rows[0]
        c = lax.axis_index("c")
        s = lax.axis_index("s")
        wid = s * NC + c

        zero16 = jnp.zeros((L,), jnp.float32)

        def zero_step(i, carry):
            for k in range(width // L):
                rows0_v[i, pl.ds(k * L, L)] = zero16
            return carry

        lax.fori_loop(0, CHUNK, zero_step, 0)
        for t in range(RPS // CHUNK):
            pltpu.sync_copy(
                rows0_v, acc_sh.at[pl.ds(s * RPS + t * CHUNK, CHUNK)]
            )
        plsc.subcore_barrier()

        n_grp = HALF // nbuf
        for h in range(N_CHUNKS // HALF):
            pltpu.sync_copy(src_hbm.at[wid, pl.ds(h * HALF, HALF)], sidx_v)
            pltpu.sync_copy(dst_hbm.at[wid, pl.ds(h * HALF, HALF)], didx_v)
            for b in range(nbuf):
                pltpu.async_copy(g_hbm.at[sidx_v.at[b]], rows[b], sems[b])

            def grp_step(t, carry):
                j0 = nbuf * t
                for b in range(nbuf):
                    j = j0 + b
                    pltpu.make_async_copy(
                        g_hbm.at[sidx_v.at[j]], rows[b], sems[b]
                    ).wait()
                    pltpu.sync_copy(rows[b], acc_sh.at[didx_v.at[j]], add=True)

                    @pl.when(t + 1 < n_grp)
                    def _():
                        pltpu.async_copy(
                            g_hbm.at[sidx_v.at[j + nbuf]], rows[b], sems[b]
                        )

                return carry

            lax.fori_loop(0, n_grp, grp_step, 0)
        plsc.subcore_barrier()

        start = s * RPS
        pltpu.sync_copy(
            acc_sh.at[pl.ds(start, RPS)], out_hbm.at[c, pl.ds(start, RPS)]
        )

    mesh = plsc.VectorSubcoreMesh(core_axis_name="c", subcore_axis_name="s")
    return pl.kernel(
        body,
        out_type=jax.ShapeDtypeStruct((NC, N_PAD, width), jnp.float32),
        mesh=mesh,
        compiler_params=pltpu.CompilerParams(
            needs_layout_passes=False,
            use_tc_tiling_on_sc=None if width % 128 == 0 else False,
        ),
        scratch_types=(
            [
                pltpu.VMEM((HALF, CHUNK), jnp.int32),
                pltpu.VMEM((HALF, CHUNK), jnp.int32),
            ]
            + [pltpu.VMEM((CHUNK, width), jnp.float32)] * nbuf
            + [pltpu.VMEM_SHARED((N_PAD, width), jnp.float32)]
            + [pltpu.SemaphoreType.DMA] * nbuf
        ),
    )


def _dis_tc_body(degp_ref, dis_ref):
    deg = jnp.sum(degp_ref[...], axis=0, keepdims=True) + 1.0
    dis_ref[...] = lax.rsqrt(deg)


def _matmul_tc_body(x_ref, w_ref, dis_ref, hcat_ref, g1_ref, xskip_ref):
    h = jnp.dot(x_ref[...], w_ref[...], preferred_element_type=jnp.float32)
    hcat_ref[...] = h[:, :D_FEAT + N_CLASSES]
    xskip_ref[...] = h[:, D_FEAT + N_CLASSES:]
    g1_ref[...] = h[:, :D_FEAT] * dis_ref[...]


def _mid_tc_body(p_ref, hcat_ref, dis_ref, w2b_ref, b1_ref, g2_ref, h2_ref):
    dis = dis_ref[...]
    p = p_ref[0] + p_ref[1]
    h1 = hcat_ref[:, :D_FEAT]
    out1 = dis * p + (dis * dis) * h1 + b1_ref[...]
    x1 = jnp.maximum(out1, 0.0)
    h2 = hcat_ref[:, D_FEAT:D_FEAT + N_CLASSES] + jnp.dot(
        x1, w2b_ref[...], preferred_element_type=jnp.float32
    )
    h2_ref[...] = h2
    g2_ref[...] = dis * h2


def _final_tc_body(q_ref, h2_ref, xskip_ref, dis_ref, b2_ref, out_ref):
    dis = dis_ref[...]
    q = q_ref[0] + q_ref[1]
    out_ref[...] = (
        dis * q + (dis * dis) * h2_ref[...] + b2_ref[...] + xskip_ref[...]
    )


_ROW_BLK = 2000
_N_BLKS = N_NODES // _ROW_BLK


def kernel(x, edge_index, W1, b1, W2, b2, W_skip):
    f32 = jnp.float32
    src = edge_index[0].astype(jnp.int32)
    dst = edge_index[1].astype(jnp.int32)
    pad = E_PAD - N_EDGES
    pad_ids = lax.iota(jnp.int32, pad)
    src_t = jnp.concatenate([src, pad_ids % N_NODES]).reshape(
        NW, N_CHUNKS, CHUNK
    )
    dst_t = jnp.concatenate([dst, DUMP + pad_ids % (N_PAD - N_NODES)]).reshape(
        NW, N_CHUNKS, CHUNK
    )
    w_cat = jnp.concatenate([W1, W2[:D_FEAT], W_skip], axis=1)
    w2b = W2[D_FEAT:]
    b1r = b1.reshape(1, HIDDEN)
    b2r = b2.reshape(1, N_CLASSES)

    degp = _make_deg_kernel()(dst_t)
    dis_row = pl.pallas_call(
        _dis_tc_body,
        out_shape=jax.ShapeDtypeStruct((1, N_PAD), f32),
    )(degp)
    dis_col = dis_row.reshape(N_PAD, 1)[:N_NODES]

    wtot = HIDDEN + 2 * N_CLASSES
    hcat, g1, xskip = pl.pallas_call(
        _matmul_tc_body,
        grid=(_N_BLKS,),
        in_specs=[
            pl.BlockSpec((_ROW_BLK, D_FEAT), lambda i: (i, 0)),
            pl.BlockSpec((D_FEAT, wtot), lambda i: (0, 0)),
            pl.BlockSpec((_ROW_BLK, 1), lambda i: (i, 0)),
        ],
        out_specs=[
            pl.BlockSpec((_ROW_BLK, HIDDEN + N_CLASSES), lambda i: (i, 0)),
            pl.BlockSpec((_ROW_BLK, D_FEAT), lambda i: (i, 0)),
            pl.BlockSpec((_ROW_BLK, N_CLASSES), lambda i: (i, 0)),
        ],
        out_shape=[
            jax.ShapeDtypeStruct((N_NODES, HIDDEN + N_CLASSES), f32),
            jax.ShapeDtypeStruct((N_NODES, D_FEAT), f32),
            jax.ShapeDtypeStruct((N_NODES, N_CLASSES), f32),
        ],
    )(x, w_cat, dis_col)

    p1 = _make_scatter_kernel(HIDDEN)(g1, src_t, dst_t)

    g2, h2 = pl.pallas_call(
        _mid_tc_body,
        grid=(_N_BLKS,),
        in_specs=[
            pl.BlockSpec((NC, _ROW_BLK, HIDDEN), lambda i: (0, i, 0)),
            pl.BlockSpec((_ROW_BLK, HIDDEN + N_CLASSES), lambda i: (i, 0)),
            pl.BlockSpec((_ROW_BLK, 1), lambda i: (i, 0)),
            pl.BlockSpec((D_FEAT, N_CLASSES), lambda i: (0, 0)),
            pl.BlockSpec((1, HIDDEN), lambda i: (0, 0)),
        ],
        out_specs=[
            pl.BlockSpec((_ROW_BLK, N_CLASSES), lambda i: (i, 0)),
            pl.BlockSpec((_ROW_BLK, N_CLASSES), lambda i: (i, 0)),
        ],
        out_shape=[
            jax.ShapeDtypeStruct((N_NODES, N_CLASSES), f32),
            jax.ShapeDtypeStruct((N_NODES, N_CLASSES), f32),
        ],
    )(p1, hcat, dis_col, w2b, b1r)

    q2 = _make_scatter_kernel(N_CLASSES)(g2, src_t, dst_t)

    out = pl.pallas_call(
        _final_tc_body,
        grid=(_N_BLKS,),
        in_specs=[
            pl.BlockSpec((NC, _ROW_BLK, N_CLASSES), lambda i: (0, i, 0)),
            pl.BlockSpec((_ROW_BLK, N_CLASSES), lambda i: (i, 0)),
            pl.BlockSpec((_ROW_BLK, N_CLASSES), lambda i: (i, 0)),
            pl.BlockSpec((_ROW_BLK, 1), lambda i: (i, 0)),
            pl.BlockSpec((1, N_CLASSES), lambda i: (0, 0)),
        ],
        out_specs=pl.BlockSpec((_ROW_BLK, N_CLASSES), lambda i: (i, 0)),
        out_shape=jax.ShapeDtypeStruct((N_NODES, N_CLASSES), f32),
    )(q2, h2, xskip, dis_col, b2r)

    return out
